# baseline (speedup 1.0000x reference)
import jax
import jax.numpy as jnp
from jax.experimental import pallas as pl

IB = 32


def _pos_enc_kernel(row_ref, col_ref, out_ref, *, w, nf):
    ce = col_ref[...]
    out_ref[:, :, 0:nf] = jnp.broadcast_to(ce[None], (IB, w, nf))
    rv = row_ref[...]
    out_ref[:, :, nf:2 * nf] = jnp.broadcast_to(rv[:, None, :], (IB, w, nf))


def kernel(bev_mask, row_embed, col_embed):
    b = bev_mask.shape[0]
    h, w = bev_mask.shape[-2], bev_mask.shape[-1]
    nf = row_embed.shape[1]

    import functools
    body = functools.partial(_pos_enc_kernel, w=w, nf=nf)

    grid = (h + IB - 1) // IB
    pos = pl.pallas_call(
        body,
        grid=(grid,),
        in_specs=[
            pl.BlockSpec((IB, nf), lambda i: (i, 0)),
            pl.BlockSpec((w, nf), lambda i: (0, 0)),
        ],
        out_specs=pl.BlockSpec((IB, w, 2 * nf), lambda i: (i, 0, 0)),
        out_shape=jax.ShapeDtypeStruct((h, w, 2 * nf), jnp.float32),
    )(row_embed[:h], col_embed[:w])
    out = jnp.transpose(pos, (2, 0, 1))[None]
    return jnp.broadcast_to(out, (b, 2 * nf, h, w))


# speedup vs baseline: 1.0003x; 1.0003x over previous
import jax
import jax.numpy as jnp
from jax.experimental import pallas as pl

IB = 16


def _pos_enc_kernel(row_ref, col_ref, out_ref, *, w, nf):
    ce = col_ref[...]
    out_ref[:, :, 0:nf] = jnp.broadcast_to(ce[None], (IB, w, nf))
    rv = row_ref[...]
    out_ref[:, :, nf:2 * nf] = jnp.broadcast_to(rv[:, None, :], (IB, w, nf))


def kernel(bev_mask, row_embed, col_embed):
    b = bev_mask.shape[0]
    h, w = bev_mask.shape[-2], bev_mask.shape[-1]
    nf = row_embed.shape[1]

    import functools
    body = functools.partial(_pos_enc_kernel, w=w, nf=nf)

    grid = (h + IB - 1) // IB
    pos = pl.pallas_call(
        body,
        grid=(grid,),
        in_specs=[
            pl.BlockSpec((IB, nf), lambda i: (i, 0)),
            pl.BlockSpec((w, nf), lambda i: (0, 0)),
        ],
        out_specs=pl.BlockSpec((IB, w, 2 * nf), lambda i: (i, 0, 0)),
        out_shape=jax.ShapeDtypeStruct((h, w, 2 * nf), jnp.float32),
    )(row_embed[:h], col_embed[:w])
    out = jnp.transpose(pos, (2, 0, 1))[None]
    return jnp.broadcast_to(out, (b, 2 * nf, h, w))
